# baseline (speedup 1.0000x reference)
import jax
import jax.numpy as jnp
from jax import lax
from jax.experimental import pallas as pl
from jax.experimental.pallas import tpu as pltpu
from jax.experimental.pallas import tpu_sc as plsc

VOCAB = 1000
BATCH = 1024
SEQ = 50
LANES = 16
NUM_CORES = 2
NUM_SUBCORES = 16
NW = NUM_CORES * NUM_SUBCORES
VSLAB = 32
VLAST = VOCAB - (NW - 1) * VSLAB
NB16 = BATCH // LANES

_mesh = plsc.VectorSubcoreMesh(core_axis_name="c", subcore_axis_name="s")


def _lookup_body(tableT_hbm, idxT_hbm, out_hbm, slab_v, idx_v, buf_v,
                 ssem, osem):
    wid = lax.axis_index("s") * NUM_CORES + lax.axis_index("c")
    v0 = wid * VSLAB
    is_last = wid == NW - 1

    @pl.when(is_last)
    def _():
        pltpu.sync_copy(tableT_hbm.at[pl.ds(v0, VLAST)],
                        slab_v.at[pl.ds(0, VLAST)])

    @pl.when(jnp.logical_not(is_last))
    def _():
        pltpu.sync_copy(tableT_hbm.at[pl.ds(v0, VSLAB)], slab_v)

    def idx_load(s, sl):
        return pltpu.make_async_copy(idxT_hbm.at[s], idx_v.at[sl], ssem)

    def out_full(s, sl):
        return pltpu.make_async_copy(
            buf_v.at[sl], out_hbm.at[s, pl.ds(v0, VSLAB), :], osem)

    def out_last(s, sl):
        return pltpu.make_async_copy(
            buf_v.at[sl, pl.ds(0, VLAST), :],
            out_hbm.at[s, pl.ds(v0, VLAST), :], osem)

    idx_load(0, 0).start()

    def body(s, carry):
        sl = lax.rem(s, 2)

        idx_load(s, sl).wait()

        @pl.when(s + 1 < SEQ)
        def _():
            idx_load(s + 1, 1 - sl).start()

        @pl.when(s >= 2)
        def _():
            @pl.when(is_last)
            def _():
                out_last(s - 2, sl).wait()

            @pl.when(jnp.logical_not(is_last))
            def _():
                out_full(s - 2, sl).wait()

        @plsc.parallel_loop(0, NB16, 1, unroll=1)
        def _(g):
            idxv = idx_v[sl, pl.ds(LANES * g, LANES)]
            for v in range(VSLAB):
                vvec = jnp.full((LANES,), v, jnp.int32)
                buf_v[sl, v, pl.ds(LANES * g, LANES)] = (
                    plsc.load_gather(slab_v, [vvec, idxv]))

        @pl.when(is_last)
        def _():
            out_last(s, sl).start()

        @pl.when(jnp.logical_not(is_last))
        def _():
            out_full(s, sl).start()

        return carry

    lax.fori_loop(0, SEQ, body, 0)

    @pl.when(is_last)
    def _():
        out_last(SEQ - 2, 0).wait()
        out_last(SEQ - 1, 1).wait()

    @pl.when(jnp.logical_not(is_last))
    def _():
        out_full(SEQ - 2, 0).wait()
        out_full(SEQ - 1, 1).wait()


_lookup = pl.kernel(
    _lookup_body,
    mesh=_mesh,
    out_type=jax.ShapeDtypeStruct((SEQ, VOCAB, BATCH), jnp.float32),
    scratch_types=[
        pltpu.VMEM((VSLAB, VOCAB), jnp.float32),
        pltpu.VMEM((2, BATCH), jnp.int32),
        pltpu.VMEM((2, VSLAB, BATCH), jnp.float32),
        pltpu.SemaphoreType.DMA,
        pltpu.SemaphoreType.DMA,
    ],
    compiler_params=pltpu.CompilerParams(needs_layout_passes=False),
)


def kernel(idx, token_embedding_table):
    tableT = token_embedding_table.T
    idxT = idx.astype(jnp.int32).T
    out_t = _lookup(tableT, idxT)
    return jnp.transpose(out_t, (2, 0, 1))


# speedup vs baseline: 1.0104x; 1.0104x over previous
import jax
import jax.numpy as jnp
from jax import lax
from jax.experimental import pallas as pl
from jax.experimental.pallas import tpu as pltpu
from jax.experimental.pallas import tpu_sc as plsc

VOCAB = 1000
BATCH = 1024
SEQ = 50
LANES = 16
NUM_CORES = 2
NUM_SUBCORES = 16
NW = NUM_CORES * NUM_SUBCORES
VSLAB = 32
VLAST = VOCAB - (NW - 1) * VSLAB
NB16 = BATCH // LANES

_mesh = plsc.VectorSubcoreMesh(core_axis_name="c", subcore_axis_name="s")


def _lookup_body(tableT_hbm, idxT_hbm, out_hbm, slab_v, idx_v, buf_v,
                 ssem, osem):
    wid = lax.axis_index("s") * NUM_CORES + lax.axis_index("c")
    v0 = wid * VSLAB
    is_last = wid == NW - 1

    @pl.when(is_last)
    def _():
        pltpu.sync_copy(tableT_hbm.at[pl.ds(v0 * VOCAB, VLAST * VOCAB)],
                        slab_v.at[pl.ds(0, VLAST * VOCAB)])

    @pl.when(jnp.logical_not(is_last))
    def _():
        pltpu.sync_copy(tableT_hbm.at[pl.ds(v0 * VOCAB, VSLAB * VOCAB)],
                        slab_v)

    def idx_load(s, sl):
        return pltpu.make_async_copy(idxT_hbm.at[s], idx_v.at[sl], ssem)

    def out_full(s, sl):
        return pltpu.make_async_copy(
            buf_v.at[sl], out_hbm.at[s, pl.ds(v0, VSLAB), :], osem)

    def out_last(s, sl):
        return pltpu.make_async_copy(
            buf_v.at[sl, pl.ds(0, VLAST), :],
            out_hbm.at[s, pl.ds(v0, VLAST), :], osem)

    idx_load(0, 0).start()

    def body(s, carry):
        sl = lax.rem(s, 2)

        idx_load(s, sl).wait()

        @pl.when(s + 1 < SEQ)
        def _():
            idx_load(s + 1, 1 - sl).start()

        @pl.when(s >= 2)
        def _():
            @pl.when(is_last)
            def _():
                out_last(s - 2, sl).wait()

            @pl.when(jnp.logical_not(is_last))
            def _():
                out_full(s - 2, sl).wait()

        @plsc.parallel_loop(0, NB16, 1, unroll=1)
        def _(g):
            idxv = idx_v[sl, pl.ds(LANES * g, LANES)]
            for v in range(VSLAB):
                buf_v[sl, v, pl.ds(LANES * g, LANES)] = (
                    plsc.load_gather(slab_v, [idxv + v * VOCAB]))

        @pl.when(is_last)
        def _():
            out_last(s, sl).start()

        @pl.when(jnp.logical_not(is_last))
        def _():
            out_full(s, sl).start()

        return carry

    lax.fori_loop(0, SEQ, body, 0)

    @pl.when(is_last)
    def _():
        out_last(SEQ - 2, 0).wait()
        out_last(SEQ - 1, 1).wait()

    @pl.when(jnp.logical_not(is_last))
    def _():
        out_full(SEQ - 2, 0).wait()
        out_full(SEQ - 1, 1).wait()


_lookup = pl.kernel(
    _lookup_body,
    mesh=_mesh,
    out_type=jax.ShapeDtypeStruct((SEQ, VOCAB, BATCH), jnp.float32),
    scratch_types=[
        pltpu.VMEM((VSLAB * VOCAB,), jnp.float32),
        pltpu.VMEM((2, BATCH), jnp.int32),
        pltpu.VMEM((2, VSLAB, BATCH), jnp.float32),
        pltpu.SemaphoreType.DMA,
        pltpu.SemaphoreType.DMA,
    ],
    compiler_params=pltpu.CompilerParams(needs_layout_passes=False),
)


def kernel(idx, token_embedding_table):
    tableT_flat = token_embedding_table.T.reshape(-1)
    idxT = idx.astype(jnp.int32).T
    out_t = _lookup(tableT_flat, idxT)
    return jnp.transpose(out_t, (2, 0, 1))
